# Initial kernel scaffold; baseline (speedup 1.0000x reference)
import jax
import jax.numpy as jnp
from jax.experimental import pallas as pl


def kernel(hidden_states, W, b):
    raise NotImplementedError("write your pallas kernel here")


# speedup vs baseline: 8.6081x; 8.6081x over previous
import jax
import jax.numpy as jnp
from jax.experimental import pallas as pl
from jax.experimental.pallas import tpu as pltpu

NUM_EXPERTS = 16
TOP_K = 2
ROW_TILE = 512


def _router_body(x_ref, w_ref, b_ref, idx_ref):
    y = jax.lax.dot_general(
        x_ref[...], w_ref[...],
        (((1,), (1,)), ((), ())),
        preferred_element_type=jnp.float32,
    )
    y = y + b_ref[...]
    bits = (y > 0).astype(jnp.int32)
    s0 = jnp.sum(bits[:, :32], axis=1, keepdims=True)
    s1 = jnp.sum(bits[:, 32:], axis=1, keepdims=True)
    r0 = jnp.bitwise_and(s0, NUM_EXPERTS - 1)
    r1 = jnp.bitwise_and(s1, NUM_EXPERTS - 1)
    i1 = jnp.where(r1 == r0, jnp.bitwise_and(r0 + 1, NUM_EXPERTS - 1), r1)
    idx_ref[...] = jnp.concatenate([r0, i1], axis=1)


def kernel(hidden_states, W, b):
    B, S, H = hidden_states.shape
    T = B * S
    x = hidden_states.reshape(T, H)
    grid = (T // ROW_TILE,)
    idx = pl.pallas_call(
        _router_body,
        grid=grid,
        in_specs=[
            pl.BlockSpec((ROW_TILE, H), lambda i: (i, 0)),
            pl.BlockSpec((TOP_K * 32, H), lambda i: (0, 0)),
            pl.BlockSpec((1, TOP_K * 32), lambda i: (0, 0)),
        ],
        out_specs=pl.BlockSpec((ROW_TILE, TOP_K), lambda i: (i, 0)),
        out_shape=jax.ShapeDtypeStruct((T, TOP_K), jnp.int32),
        compiler_params=pltpu.CompilerParams(
            dimension_semantics=("arbitrary",),
        ),
    )(x, W, b.reshape(1, TOP_K * 32))
    expert_indices = idx.astype(jnp.int64)
    expert_weights = jnp.full((T, TOP_K), 1.0 / TOP_K, dtype=jnp.float32)
    router_logits = jnp.zeros((T, NUM_EXPERTS), dtype=jnp.float32)
    return (expert_weights, expert_indices, router_logits)


# speedup vs baseline: 9.6667x; 1.1230x over previous
import jax
import jax.numpy as jnp
from jax.experimental import pallas as pl
from jax.experimental.pallas import tpu as pltpu

NUM_EXPERTS = 16
TOP_K = 2
ROW_TILE = 1024


def _router_body(x_ref, w_ref, b_ref, idx_ref):
    y = jax.lax.dot_general(
        x_ref[...], w_ref[...],
        (((1,), (1,)), ((), ())),
        preferred_element_type=jnp.float32,
    )
    y = y + b_ref[...]
    bits = (y > 0).astype(jnp.int32)
    s0 = jnp.sum(bits[:, :32], axis=1, keepdims=True)
    s1 = jnp.sum(bits[:, 32:], axis=1, keepdims=True)
    r0 = jnp.bitwise_and(s0, NUM_EXPERTS - 1)
    r1 = jnp.bitwise_and(s1, NUM_EXPERTS - 1)
    i1 = jnp.where(r1 == r0, jnp.bitwise_and(r0 + 1, NUM_EXPERTS - 1), r1)
    idx_ref[...] = jnp.concatenate([r0, i1], axis=1)


def kernel(hidden_states, W, b):
    B, S, H = hidden_states.shape
    T = B * S
    x = hidden_states.reshape(T, H)
    grid = (T // ROW_TILE,)
    idx = pl.pallas_call(
        _router_body,
        grid=grid,
        in_specs=[
            pl.BlockSpec((ROW_TILE, H), lambda i: (i, 0)),
            pl.BlockSpec((TOP_K * 32, H), lambda i: (0, 0)),
            pl.BlockSpec((1, TOP_K * 32), lambda i: (0, 0)),
        ],
        out_specs=pl.BlockSpec((ROW_TILE, TOP_K), lambda i: (i, 0)),
        out_shape=jax.ShapeDtypeStruct((T, TOP_K), jnp.int32),
        compiler_params=pltpu.CompilerParams(
            dimension_semantics=("arbitrary",),
        ),
    )(x, W, b.reshape(1, TOP_K * 32))
    expert_indices = idx.astype(jnp.int64)
    expert_weights = jnp.full((T, TOP_K), 1.0 / TOP_K, dtype=jnp.float32)
    router_logits = jnp.zeros((T, NUM_EXPERTS), dtype=jnp.float32)
    return (expert_weights, expert_indices, router_logits)
